# baseline (speedup 1.0000x reference)
import functools

import jax
import jax.numpy as jnp
from jax import lax
from jax.experimental import pallas as pl
from jax.experimental.pallas import tpu as pltpu
from jax.experimental.pallas import tpu_sc as plsc

_LANES = 16


def kernel(x, pe_table):
    S, B, D = x.shape
    BD = B * D
    info = plsc.get_sparse_core_info()
    NC, NS = info.num_cores, info.num_subcores
    NW = NC * NS
    RPW = S // NW
    CH = 8
    NCH = RPW // CH
    mesh = plsc.VectorSubcoreMesh(core_axis_name="c", subcore_axis_name="s")

    @functools.partial(
        pl.kernel,
        mesh=mesh,
        out_type=jax.ShapeDtypeStruct((S, BD), jnp.float32),
        scratch_types=[
            pltpu.VMEM((2, CH, BD), jnp.float32),
            pltpu.VMEM((2, CH, D), jnp.float32),
            pltpu.SemaphoreType.DMA,
            pltpu.SemaphoreType.DMA,
            pltpu.SemaphoreType.DMA,
            pltpu.SemaphoreType.DMA,
        ],
    )
    def sc_add(x_hbm, pe_hbm, out_hbm, x_buf, pe_buf, si0, si1, so0, so1):
        wid = lax.axis_index("s") * NC + lax.axis_index("c")
        base = wid * RPW
        s_in = (si0, si1)
        s_out = (so0, so1)
        in_h = [None, None]
        out_h = [None, None]

        def start_in(c):
            b = c % 2
            rs = base + c * CH
            hx = pltpu.async_copy(
                x_hbm.at[pl.ds(rs, CH)], x_buf.at[b], s_in[b])
            hp = pltpu.async_copy(
                pe_hbm.at[pl.ds(rs, CH)], pe_buf.at[b], s_in[b])
            in_h[b] = (hx, hp)

        start_in(0)
        for c in range(NCH):
            b = c % 2
            if c + 1 < NCH:
                if out_h[1 - b] is not None:
                    out_h[1 - b].wait()
                    out_h[1 - b] = None
                start_in(c + 1)
            for h in in_h[b]:
                h.wait()

            def jbody(j, carry, b=b):
                for r in range(CH):
                    pe_v = pe_buf[b, r, pl.ds(j * _LANES, _LANES)]
                    for bb in range(B):
                        plsc.addupdate(
                            x_buf.at[b, r, pl.ds(bb * D + j * _LANES, _LANES)],
                            pe_v)
                return carry

            lax.fori_loop(0, D // _LANES, jbody, 0)
            rs = base + c * CH
            out_h[b] = pltpu.async_copy(
                x_buf.at[b], out_hbm.at[pl.ds(rs, CH)], s_out[b])
        for h in out_h:
            if h is not None:
                h.wait()

    out2d = sc_add(x.reshape(S, BD), pe_table[:S])
    return out2d.reshape(S, B, D)


# speedup vs baseline: 2.5396x; 2.5396x over previous
import functools

import jax
import jax.numpy as jnp
from jax import lax
from jax.experimental import pallas as pl
from jax.experimental.pallas import tpu as pltpu
from jax.experimental.pallas import tpu_sc as plsc

_LANES = 16


def kernel(x, pe_table):
    S, B, D = x.shape
    info = plsc.get_sparse_core_info()
    NC, NS = info.num_cores, info.num_subcores
    NW = NC * NS
    RPW = S // NW
    CH = 8
    NCH = RPW // CH
    mesh = plsc.VectorSubcoreMesh(core_axis_name="c", subcore_axis_name="s")

    @functools.partial(
        pl.kernel,
        mesh=mesh,
        compiler_params=pltpu.CompilerParams(use_tc_tiling_on_sc=True),
        out_type=jax.ShapeDtypeStruct((S, B, D), jnp.float32),
        scratch_types=[
            pltpu.VMEM((2, CH, B, D), jnp.float32),
            pltpu.VMEM((2, CH, D), jnp.float32),
            pltpu.SemaphoreType.DMA,
            pltpu.SemaphoreType.DMA,
            pltpu.SemaphoreType.DMA,
            pltpu.SemaphoreType.DMA,
        ],
    )
    def sc_add(x_hbm, pe_hbm, out_hbm, x_buf, pe_buf, si0, si1, so0, so1):
        wid = lax.axis_index("s") * NC + lax.axis_index("c")
        base = wid * RPW
        s_in = (si0, si1)
        s_out = (so0, so1)
        in_h = [None, None]
        out_h = [None, None]

        def start_in(c):
            b = c % 2
            rs = base + c * CH
            hx = pltpu.async_copy(
                x_hbm.at[pl.ds(rs, CH)], x_buf.at[b], s_in[b])
            hp = pltpu.async_copy(
                pe_hbm.at[pl.ds(rs, CH)], pe_buf.at[b], s_in[b])
            in_h[b] = (hx, hp)

        start_in(0)
        for c in range(NCH):
            b = c % 2
            if c + 1 < NCH:
                if out_h[1 - b] is not None:
                    out_h[1 - b].wait()
                    out_h[1 - b] = None
                start_in(c + 1)
            for h in in_h[b]:
                h.wait()

            def jbody(j, carry, b=b):
                for r in range(CH):
                    pe_v = pe_buf[b, r, pl.ds(j * _LANES, _LANES)]
                    for bb in range(B):
                        plsc.addupdate(
                            x_buf.at[b, r, bb, pl.ds(j * _LANES, _LANES)],
                            pe_v)
                return carry

            lax.fori_loop(0, D // _LANES, jbody, 0)
            rs = base + c * CH
            out_h[b] = pltpu.async_copy(
                x_buf.at[b], out_hbm.at[pl.ds(rs, CH)], s_out[b])
        for h in out_h:
            if h is not None:
                h.wait()

    return sc_add(x, pe_table[:S])
